# Initial kernel scaffold; baseline (speedup 1.0000x reference)
import jax
import jax.numpy as jnp
from jax.experimental import pallas as pl


def kernel(x, edge_index, edge_attr, W_l, b_l, W_r, b_r, W_e, att, bias, gW1, gb1, gW2, gb2):
    raise NotImplementedError("write your pallas kernel here")


# speedup vs baseline: 4.2613x; 4.2613x over previous
import jax
import jax.numpy as jnp
from jax.experimental import pallas as pl
from jax.experimental.pallas import tpu as pltpu

N = 10000
E = 160000
HEADS = 4
OUT = 256
HO = HEADS * OUT
NEG = 0.2
EP = 172032


def _mm_bias_kernel(a_ref, w_ref, b_ref, o_ref):
    o_ref[...] = (
        jnp.dot(a_ref[...], w_ref[...], preferred_element_type=jnp.float32)
        + b_ref[...]
    )


def _mm_bias(a, w, b, bm):
    m, k = a.shape
    n = w.shape[1]
    return pl.pallas_call(
        _mm_bias_kernel,
        grid=(m // bm,),
        in_specs=[
            pl.BlockSpec((bm, k), lambda i: (i, 0)),
            pl.BlockSpec((k, n), lambda i: (0, 0)),
            pl.BlockSpec((1, n), lambda i: (0, 0)),
        ],
        out_specs=pl.BlockSpec((bm, n), lambda i: (i, 0)),
        out_shape=jax.ShapeDtypeStruct((m, n), jnp.float32),
    )(a, w, b.reshape(1, n))


def _alpha_kernel(xj_ref, xi_ref, ee_ref, a_ref, o_ref):
    m = xj_ref[...] + xi_ref[...] + ee_ref[...]
    m = jnp.where(m > 0, m, NEG * m)
    o_ref[...] = jnp.exp(
        jnp.dot(m, a_ref[...], preferred_element_type=jnp.float32)
    )


def _alpha(xj, xi, ee, a_sel, bm):
    return pl.pallas_call(
        _alpha_kernel,
        grid=(EP // bm,),
        in_specs=[
            pl.BlockSpec((bm, HO), lambda i: (i, 0)),
            pl.BlockSpec((bm, HO), lambda i: (i, 0)),
            pl.BlockSpec((bm, HO), lambda i: (i, 0)),
            pl.BlockSpec((HO, 8), lambda i: (0, 0)),
        ],
        out_specs=pl.BlockSpec((bm, 8), lambda i: (i, 0)),
        out_shape=jax.ShapeDtypeStruct((EP, 8), jnp.float32),
    )(xj, xi, ee, a_sel)


def _wfold_kernel(xj_ref, a_ref, d_ref, o_ref):
    acc = jnp.zeros(o_ref.shape, jnp.float32)
    for h in range(HEADS):
        w = a_ref[:, h : h + 1] * 0.25 / (d_ref[:, h : h + 1] + 1e-16)
        acc = acc + xj_ref[:, h * OUT : (h + 1) * OUT] * w
    o_ref[...] = acc


def _wfold(xj, alpha8, dg, bm):
    return pl.pallas_call(
        _wfold_kernel,
        grid=(EP // bm,),
        in_specs=[
            pl.BlockSpec((bm, HO), lambda i: (i, 0)),
            pl.BlockSpec((bm, 8), lambda i: (i, 0)),
            pl.BlockSpec((bm, 8), lambda i: (i, 0)),
        ],
        out_specs=pl.BlockSpec((bm, OUT), lambda i: (i, 0)),
        out_shape=jax.ShapeDtypeStruct((EP, OUT), jnp.float32),
    )(xj, alpha8, dg)


def kernel(x, edge_index, edge_attr, W_l, b_l, W_r, b_r, W_e, att, bias,
           gW1, gb1, gW2, gb2):
    f32 = jnp.float32
    src = edge_index[0]
    dst = edge_index[1]

    x_l = _mm_bias(x, W_l, b_l, 1000)
    x_r = _mm_bias(x, W_r, b_r, 1000)

    ones = jnp.ones((E,), f32)
    deg = jax.ops.segment_sum(ones, dst, num_segments=N)
    ea_sum = jax.ops.segment_sum(edge_attr, dst, num_segments=N)
    loop_attr = ea_sum / jnp.clip(deg, 1.0, None)[:, None]

    ea_f = jnp.concatenate(
        [edge_attr, loop_attr, jnp.zeros((EP - E - N, 16), f32)], axis=0
    )
    ee = _mm_bias(ea_f, W_e, jnp.zeros((HO,), f32), 1024)

    loop = jnp.arange(N, dtype=src.dtype)
    pad = jnp.zeros((EP - E - N,), src.dtype)
    srcp = jnp.concatenate([src, loop, pad])
    dstp = jnp.concatenate([dst, loop, jnp.full((EP - E - N,), N, src.dtype)])
    dstp_c = jnp.minimum(dstp, N - 1)

    xj = jnp.take(x_l, srcp, axis=0)
    xi = jnp.take(x_r, dstp_c, axis=0)

    lanes = jnp.arange(HO)
    a_sel = jnp.zeros((HO, 8), f32).at[lanes, lanes // OUT].set(
        att.reshape(-1)
    )
    alpha8 = _alpha(xj, xi, ee, a_sel, 512)

    denom = jax.ops.segment_sum(alpha8, dstp, num_segments=N + 1)
    dg = jnp.take(denom, dstp, axis=0)

    wx = _wfold(xj, alpha8, dg, 512)
    out = jax.ops.segment_sum(wx, dstp, num_segments=N + 1)[:N]
    return out + bias


# speedup vs baseline: 5.7757x; 1.3554x over previous
import functools

import jax
import jax.numpy as jnp
from jax import lax
from jax.experimental import pallas as pl
from jax.experimental.pallas import tpu as pltpu
from jax.experimental.pallas import tpu_sc as plsc

N = 10000
E = 160000
HEADS = 4
OUT = 256
HO = HEADS * OUT
NEG = 0.2
EP = 172032


def _mm_bias_kernel(a_ref, w_ref, b_ref, o_ref):
    o_ref[...] = (
        jnp.dot(a_ref[...], w_ref[...], preferred_element_type=jnp.float32)
        + b_ref[...]
    )


def _mm_bias(a, w, b, bm):
    m, k = a.shape
    n = w.shape[1]
    return pl.pallas_call(
        _mm_bias_kernel,
        grid=(m // bm,),
        in_specs=[
            pl.BlockSpec((bm, k), lambda i: (i, 0)),
            pl.BlockSpec((k, n), lambda i: (0, 0)),
            pl.BlockSpec((1, n), lambda i: (0, 0)),
        ],
        out_specs=pl.BlockSpec((bm, n), lambda i: (i, 0)),
        out_shape=jax.ShapeDtypeStruct((m, n), jnp.float32),
    )(a, w, b.reshape(1, n))


def _alpha_kernel(xj_ref, xi_ref, ee_ref, a_ref, o_ref):
    m = xj_ref[...] + xi_ref[...] + ee_ref[...]
    m = jnp.where(m > 0, m, NEG * m)
    o_ref[...] = jnp.exp(
        jnp.dot(m, a_ref[...], preferred_element_type=jnp.float32)
    )


def _alpha(xj, xi, ee, a_sel, bm):
    return pl.pallas_call(
        _alpha_kernel,
        grid=(EP // bm,),
        in_specs=[
            pl.BlockSpec((bm, HO), lambda i: (i, 0)),
            pl.BlockSpec((bm, HO), lambda i: (i, 0)),
            pl.BlockSpec((bm, HO), lambda i: (i, 0)),
            pl.BlockSpec((HO, 8), lambda i: (0, 0)),
        ],
        out_specs=pl.BlockSpec((bm, 8), lambda i: (i, 0)),
        out_shape=jax.ShapeDtypeStruct((EP, 8), jnp.float32),
    )(xj, xi, ee, a_sel)


def _wfold_kernel(xj_ref, a_ref, d_ref, o_ref):
    acc = jnp.zeros(o_ref.shape, jnp.float32)
    for h in range(HEADS):
        w = a_ref[:, h : h + 1] * 0.25 / (d_ref[:, h : h + 1] + 1e-16)
        acc = acc + xj_ref[:, h * OUT : (h + 1) * OUT] * w
    o_ref[...] = acc


def _wfold(xj, alpha8, dg, bm):
    return pl.pallas_call(
        _wfold_kernel,
        grid=(EP // bm,),
        in_specs=[
            pl.BlockSpec((bm, HO), lambda i: (i, 0)),
            pl.BlockSpec((bm, 8), lambda i: (i, 0)),
            pl.BlockSpec((bm, 128), lambda i: (i, 0)),
        ],
        out_specs=pl.BlockSpec((bm, OUT), lambda i: (i, 0)),
        out_shape=jax.ShapeDtypeStruct((EP, OUT), jnp.float32),
    )(xj, alpha8, dg)


_NW = 32


def _sc_gather(table, idx, c):
    b, = idx.shape
    d = table.shape[1]
    b_per_w = b // _NW
    iters = b_per_w // c
    mesh = plsc.VectorSubcoreMesh(core_axis_name="c", subcore_axis_name="s")

    @functools.partial(
        pl.kernel,
        mesh=mesh,
        out_type=jax.ShapeDtypeStruct((b, d), jnp.float32),
        scratch_types=[
            pltpu.VMEM((c,), jnp.int32),
            pltpu.VMEM((c, d), jnp.float32),
            pltpu.SemaphoreType.DMA,
        ],
    )
    def k(table_hbm, idx_hbm, out_hbm, idx_v, rows_v, sem):
        wid = lax.axis_index("s") * 2 + lax.axis_index("c")
        base = wid * b_per_w

        def body(i, carry):
            off = base + i * c
            pltpu.sync_copy(idx_hbm.at[pl.ds(off, c)], idx_v)
            pltpu.async_copy(table_hbm.at[idx_v], rows_v, sem).wait()
            pltpu.sync_copy(rows_v, out_hbm.at[pl.ds(off, c)])
            return carry

        lax.fori_loop(0, iters, body, 0)

    return k(table, idx)


def kernel(x, edge_index, edge_attr, W_l, b_l, W_r, b_r, W_e, att, bias,
           gW1, gb1, gW2, gb2):
    f32 = jnp.float32
    src = edge_index[0]
    dst = edge_index[1]

    x_l = _mm_bias(x, W_l, b_l, 1000)
    x_r = _mm_bias(x, W_r, b_r, 1000)

    ones = jnp.ones((E,), f32)
    deg = jax.ops.segment_sum(ones, dst, num_segments=N)
    ea_sum = jax.ops.segment_sum(edge_attr, dst, num_segments=N)
    loop_attr = ea_sum / jnp.clip(deg, 1.0, None)[:, None]

    ea_f = jnp.concatenate(
        [edge_attr, loop_attr, jnp.zeros((EP - E - N, 16), f32)], axis=0
    )
    ee = _mm_bias(ea_f, W_e, jnp.zeros((HO,), f32), 1024)

    loop = jnp.arange(N, dtype=src.dtype)
    pad = jnp.zeros((EP - E - N,), src.dtype)
    srcp = jnp.concatenate([src, loop, pad])
    dstp = jnp.concatenate([dst, loop, jnp.full((EP - E - N,), N, src.dtype)])
    dstp_c = jnp.minimum(dstp, N - 1)

    xj = _sc_gather(x_l, srcp, 96)
    xi = _sc_gather(x_r, dstp_c, 96)

    lanes = jnp.arange(HO)
    a_sel = jnp.zeros((HO, 8), f32).at[lanes, lanes // OUT].set(
        att.reshape(-1)
    )
    alpha8 = _alpha(xj, xi, ee, a_sel, 512)

    denom = jax.ops.segment_sum(alpha8, dstp, num_segments=N + 1)
    denom_t = jnp.pad(denom, ((0, 0), (0, 120)))
    dg = _sc_gather(denom_t, dstp, 96)

    wx = _wfold(xj, alpha8, dg, 512)
    out = jax.ops.segment_sum(wx, dstp, num_segments=N + 1)[:N]
    return out + bias


# speedup vs baseline: 6.0441x; 1.0465x over previous
import functools

import jax
import jax.numpy as jnp
from jax import lax
from jax.experimental import pallas as pl
from jax.experimental.pallas import tpu as pltpu
from jax.experimental.pallas import tpu_sc as plsc

N = 10000
E = 160000
HEADS = 4
OUT = 256
HO = HEADS * OUT
NEG = 0.2
EP = 172032


def _mm_bias_kernel(a_ref, w_ref, b_ref, o_ref):
    o_ref[...] = (
        jnp.dot(a_ref[...], w_ref[...], preferred_element_type=jnp.float32)
        + b_ref[...]
    )


def _mm_bias(a, w, b, bm):
    m, k = a.shape
    n = w.shape[1]
    return pl.pallas_call(
        _mm_bias_kernel,
        grid=(m // bm,),
        in_specs=[
            pl.BlockSpec((bm, k), lambda i: (i, 0)),
            pl.BlockSpec((k, n), lambda i: (0, 0)),
            pl.BlockSpec((1, n), lambda i: (0, 0)),
        ],
        out_specs=pl.BlockSpec((bm, n), lambda i: (i, 0)),
        out_shape=jax.ShapeDtypeStruct((m, n), jnp.float32),
    )(a, w, b.reshape(1, n))


def _alpha_kernel(xj_ref, xi_ref, ee_ref, a_ref, o_ref):
    m = xj_ref[...] + xi_ref[...] + ee_ref[...]
    m = jnp.where(m > 0, m, NEG * m)
    o_ref[...] = jnp.exp(
        jnp.dot(m, a_ref[...], preferred_element_type=jnp.float32)
    )


def _alpha(xj, xi, ee, a_sel, bm):
    return pl.pallas_call(
        _alpha_kernel,
        grid=(EP // bm,),
        in_specs=[
            pl.BlockSpec((bm, HO), lambda i: (i, 0)),
            pl.BlockSpec((bm, HO), lambda i: (i, 0)),
            pl.BlockSpec((bm, HO), lambda i: (i, 0)),
            pl.BlockSpec((HO, 8), lambda i: (0, 0)),
        ],
        out_specs=pl.BlockSpec((bm, 8), lambda i: (i, 0)),
        out_shape=jax.ShapeDtypeStruct((EP, 8), jnp.float32),
    )(xj, xi, ee, a_sel)


def _wfold_kernel(xj_ref, a_ref, d_ref, o_ref):
    acc = jnp.zeros(o_ref.shape, jnp.float32)
    for h in range(HEADS):
        w = a_ref[:, h : h + 1] * 0.25 / (d_ref[:, h : h + 1] + 1e-16)
        acc = acc + xj_ref[:, h * OUT : (h + 1) * OUT] * w
    o_ref[...] = acc


def _wfold(xj, alpha8, dg, bm):
    return pl.pallas_call(
        _wfold_kernel,
        grid=(EP // bm,),
        in_specs=[
            pl.BlockSpec((bm, HO), lambda i: (i, 0)),
            pl.BlockSpec((bm, 8), lambda i: (i, 0)),
            pl.BlockSpec((bm, 128), lambda i: (i, 0)),
        ],
        out_specs=pl.BlockSpec((bm, OUT), lambda i: (i, 0)),
        out_shape=jax.ShapeDtypeStruct((EP, OUT), jnp.float32),
    )(xj, alpha8, dg)


_NW = 32


def _sc_gather(table, idx, c):
    b, = idx.shape
    d = table.shape[1]
    b_per_w = b // _NW
    iters = b_per_w // c
    mesh = plsc.VectorSubcoreMesh(core_axis_name="c", subcore_axis_name="s")

    @functools.partial(
        pl.kernel,
        mesh=mesh,
        out_type=jax.ShapeDtypeStruct((b, d), jnp.float32),
        scratch_types=[
            pltpu.VMEM((2, c), jnp.int32),
            pltpu.VMEM((2, c, d), jnp.float32),
            pltpu.SemaphoreType.DMA,
        ],
    )
    def k(table_hbm, idx_hbm, out_hbm, idx_v, rows_v, sem):
        wid = lax.axis_index("s") * 2 + lax.axis_index("c")
        base = wid * b_per_w

        def start(i, b2):
            pltpu.sync_copy(idx_hbm.at[pl.ds(base + i * c, c)],
                            idx_v.at[b2])
            pltpu.async_copy(table_hbm.at[idx_v.at[b2]], rows_v.at[b2], sem)

        def finish(i, b2):
            pltpu.make_async_copy(
                table_hbm.at[idx_v.at[b2]], rows_v.at[b2], sem
            ).wait()
            pltpu.sync_copy(rows_v.at[b2], out_hbm.at[pl.ds(base + i * c, c)])

        start(0, 0)

        def body(t, carry):
            for b2 in range(2):
                i = t * 2 + b2

                @pl.when(i + 1 < iters)
                def _():
                    start(i + 1, 1 - b2)

                finish(i, b2)
            return carry

        lax.fori_loop(0, iters // 2, body, 0)

    return k(table, idx)


def kernel(x, edge_index, edge_attr, W_l, b_l, W_r, b_r, W_e, att, bias,
           gW1, gb1, gW2, gb2):
    f32 = jnp.float32
    src = edge_index[0]
    dst = edge_index[1]

    x_l = _mm_bias(x, W_l, b_l, 1000)
    x_r = _mm_bias(x, W_r, b_r, 1000)

    ones = jnp.ones((E,), f32)
    deg = jax.ops.segment_sum(ones, dst, num_segments=N)
    ea_sum = jax.ops.segment_sum(edge_attr, dst, num_segments=N)
    loop_attr = ea_sum / jnp.clip(deg, 1.0, None)[:, None]

    ea_f = jnp.concatenate(
        [edge_attr, loop_attr, jnp.zeros((EP - E - N, 16), f32)], axis=0
    )
    ee = _mm_bias(ea_f, W_e, jnp.zeros((HO,), f32), 1024)

    loop = jnp.arange(N, dtype=src.dtype)
    pad = jnp.zeros((EP - E - N,), src.dtype)
    srcp = jnp.concatenate([src, loop, pad])
    dstp = jnp.concatenate([dst, loop, jnp.full((EP - E - N,), N, src.dtype)])
    dstp_c = jnp.minimum(dstp, N - 1)

    xj = _sc_gather(x_l, srcp, 48)
    xi = _sc_gather(x_r, dstp_c, 48)

    lanes = jnp.arange(HO)
    a_sel = jnp.zeros((HO, 8), f32).at[lanes, lanes // OUT].set(
        att.reshape(-1)
    )
    alpha8 = _alpha(xj, xi, ee, a_sel, 512)

    denom = jax.ops.segment_sum(alpha8, dstp, num_segments=N + 1)
    denom_t = jnp.pad(denom, ((0, 0), (0, 120)))
    dg = _sc_gather(denom_t, dstp, 96)

    wx = _wfold(xj, alpha8, dg, 512)
    out = jax.ops.segment_sum(wx, dstp, num_segments=N + 1)[:N]
    return out + bias
